# baseline (speedup 1.0000x reference)
import numpy as np
import jax
import jax.numpy as jnp
from jax.experimental import pallas as pl
from jax.experimental.pallas import tpu as pltpu

_S = 16
_H = 50
_W = 50
_K = 768
_O = 512
_G = 10
_LW = 64

_S2_NP = np.zeros((_W * _S, _S * _LW), dtype=np.float32)
_m = np.arange(_W * _S)
_S2_NP[_m, (_m % _S) * _LW + _m // _S] = 1.0

_S3T_NP = np.zeros((_K, _K), dtype=np.float32)
_k = np.arange(_K)
_c, _dy, _dx = _k // (_S * _S), (_k // _S) % _S, _k % _S
_S3T_NP[_dx * 48 + _c * _S + _dy, _k] = 1.0


def _conv_kernel(x_ref, s2_ref, s3t_ref, wt_ref, b_ref, o_ref, wp_ref):
    @pl.when(pl.program_id(0) == 0)
    def _permute_weights():
        wp_ref[...] = jnp.dot(
            s3t_ref[...], wt_ref[...].astype(jnp.bfloat16),
            preferred_element_type=jnp.float32,
        ).astype(jnp.bfloat16)

    X = x_ref[0].astype(jnp.bfloat16)
    X2 = X.reshape(3 * _G * _S, _W * _S)
    X3 = jnp.dot(X2, s2_ref[...],
                 preferred_element_type=jnp.float32).astype(jnp.bfloat16)
    cols = []
    for ii in range(_G):
        X3i = jnp.concatenate(
            [X3[c * _G * _S + ii * _S: c * _G * _S + ii * _S + _S, :]
             for c in range(3)], axis=0)
        cols.append(jnp.concatenate(
            [X3i[:, dx * _LW:(dx + 1) * _LW] for dx in range(_S)],
            axis=0))
    P = jnp.concatenate(cols, axis=1)
    Y = jax.lax.dot_general(
        P, wp_ref[...], (((0,), (0,)), ((), ())),
        preferred_element_type=jnp.float32,
    ) + b_ref[...]
    for ii in range(_G):
        Yi = Y[ii * _LW: ii * _LW + _W, :]
        V = jnp.stack([Yi[:, oc * 128:(oc + 1) * 128] for oc in range(4)],
                      axis=1).reshape(_W * 4, 128)
        o_ref[pl.ds(ii * _W * 4, _W * 4), :] = V


def kernel(x, W_ext, b_ext, W_conv1, b_conv1, W_loc, b_loc, W_score, b_score):
    s2 = jnp.asarray(_S2_NP, dtype=jnp.bfloat16)
    s3t = jnp.asarray(_S3T_NP, dtype=jnp.bfloat16)
    w_t = W_ext.reshape(_O, _K).T
    bias = b_ext.reshape(1, _O)

    out = pl.pallas_call(
        _conv_kernel,
        grid=(pl.cdiv(_H, _G),),
        in_specs=[
            pl.BlockSpec((1, 3, _G * _S, _W * _S), lambda n: (0, 0, n, 0)),
            pl.BlockSpec((_W * _S, _S * _LW), lambda n: (0, 0)),
            pl.BlockSpec((_K, _K), lambda n: (0, 0)),
            pl.BlockSpec((_K, _O), lambda n: (0, 0)),
            pl.BlockSpec((1, _O), lambda n: (0, 0)),
        ],
        out_specs=pl.BlockSpec((_G * _W * 4, 128), lambda n: (n, 0)),
        out_shape=jax.ShapeDtypeStruct((_H * _W * 4, 128), jnp.float32),
        scratch_shapes=[pltpu.VMEM((_K, _O), jnp.bfloat16)],
    )(x, s2, s3t, w_t, bias)

    return (out.reshape(_H, _W, _O)
            .transpose(2, 0, 1).reshape(1, _O, _H, _W))


# speedup vs baseline: 1.2686x; 1.2686x over previous
import numpy as np
import jax
import jax.numpy as jnp
from jax.experimental import pallas as pl
from jax.experimental.pallas import tpu as pltpu

_S = 16
_H = 50
_W = 50
_K = 768
_O = 512
_G = 5
_LW = 64

_S2_NP = np.zeros((_W * _S, _S * _LW), dtype=np.float32)
_m = np.arange(_W * _S)
_S2_NP[_m, (_m % _S) * _LW + _m // _S] = 1.0

_S3T_NP = np.zeros((_K, _K), dtype=np.float32)
_k = np.arange(_K)
_c, _dy, _dx = _k // (_S * _S), (_k // _S) % _S, _k % _S
_S3T_NP[_dx * 48 + _c * _S + _dy, _k] = 1.0


def _conv_kernel(x_ref, s2_ref, s3t_ref, wt_ref, b_ref, o_ref, wp_ref):
    @pl.when(pl.program_id(0) == 0)
    def _permute_weights():
        wp_ref[...] = jnp.dot(
            s3t_ref[...], wt_ref[...].astype(jnp.bfloat16),
            preferred_element_type=jnp.float32,
        ).astype(jnp.bfloat16)

    X = x_ref[0].astype(jnp.bfloat16)
    X2 = X.reshape(3 * _G * _S, _W * _S)
    X3 = jnp.dot(X2, s2_ref[...],
                 preferred_element_type=jnp.float32).astype(jnp.bfloat16)
    cols = []
    for ii in range(_G):
        X3i = jnp.concatenate(
            [X3[c * _G * _S + ii * _S: c * _G * _S + ii * _S + _S, :]
             for c in range(3)], axis=0)
        cols.append(jnp.concatenate(
            [X3i[:, dx * _LW:(dx + 1) * _LW] for dx in range(_S)],
            axis=0))
    P = jnp.concatenate(cols, axis=1)
    Y = jax.lax.dot_general(
        P, wp_ref[...], (((0,), (0,)), ((), ())),
        preferred_element_type=jnp.float32,
    ) + b_ref[...]
    for ii in range(_G):
        o_ref[ii, :, :] = Y[ii * _LW: ii * _LW + _W, :]


def kernel(x, W_ext, b_ext, W_conv1, b_conv1, W_loc, b_loc, W_score, b_score):
    s2 = jnp.asarray(_S2_NP, dtype=jnp.bfloat16)
    s3t = jnp.asarray(_S3T_NP, dtype=jnp.bfloat16)
    w_t = W_ext.reshape(_O, _K).T
    bias = b_ext.reshape(1, _O)

    out = pl.pallas_call(
        _conv_kernel,
        grid=(pl.cdiv(_H, _G),),
        in_specs=[
            pl.BlockSpec((1, 3, _G * _S, _W * _S), lambda n: (0, 0, n, 0)),
            pl.BlockSpec((_W * _S, _S * _LW), lambda n: (0, 0)),
            pl.BlockSpec((_K, _K), lambda n: (0, 0)),
            pl.BlockSpec((_K, _O), lambda n: (0, 0)),
            pl.BlockSpec((1, _O), lambda n: (0, 0)),
        ],
        out_specs=pl.BlockSpec((_G, _W, _O), lambda n: (n, 0, 0)),
        out_shape=jax.ShapeDtypeStruct((_H, _W, _O), jnp.float32),
        scratch_shapes=[pltpu.VMEM((_K, _O), jnp.bfloat16)],
    )(x, s2, s3t, w_t, bias)

    return out.transpose(2, 0, 1).reshape(1, _O, _H, _W)


# speedup vs baseline: 1.4368x; 1.1326x over previous
import numpy as np
import jax
import jax.numpy as jnp
from jax.experimental import pallas as pl
from jax.experimental.pallas import tpu as pltpu

_S = 16
_H = 50
_W = 50
_K = 768
_O = 512
_G = 10
_LW = 64

_S2_NP = np.zeros((_W * _S, _S * _LW), dtype=np.float32)
_m = np.arange(_W * _S)
_S2_NP[_m, (_m % _S) * _LW + _m // _S] = 1.0

_S3T_NP = np.zeros((_K, _K), dtype=np.float32)
_k = np.arange(_K)
_c, _dy, _dx = _k // (_S * _S), (_k // _S) % _S, _k % _S
_S3T_NP[_dx * 48 + _c * _S + _dy, _k] = 1.0


def _conv_kernel(x_ref, s2_ref, s3t_ref, wt_ref, b_ref, o_ref, wp_ref):
    @pl.when(pl.program_id(0) == 0)
    def _permute_weights():
        wp_ref[...] = jnp.dot(
            s3t_ref[...], wt_ref[...].astype(jnp.bfloat16),
            preferred_element_type=jnp.float32,
        ).astype(jnp.bfloat16)

    X = x_ref[0].astype(jnp.bfloat16)
    X2 = X.reshape(3 * _G * _S, _W * _S)
    X3 = jnp.dot(X2, s2_ref[...],
                 preferred_element_type=jnp.float32).astype(jnp.bfloat16)
    cols = []
    for ii in range(_G):
        X3i = jnp.concatenate(
            [X3[c * _G * _S + ii * _S: c * _G * _S + ii * _S + _S, :]
             for c in range(3)], axis=0)
        cols.append(jnp.concatenate(
            [X3i[:, dx * _LW:(dx + 1) * _LW] for dx in range(_S)],
            axis=0))
    P = jnp.concatenate(cols, axis=1)
    Y = jax.lax.dot_general(
        P, wp_ref[...], (((0,), (0,)), ((), ())),
        preferred_element_type=jnp.float32,
    ) + b_ref[...]
    for ii in range(_G):
        o_ref[ii, :, :] = Y[ii * _LW: ii * _LW + _W, :]


def kernel(x, W_ext, b_ext, W_conv1, b_conv1, W_loc, b_loc, W_score, b_score):
    s2 = jnp.asarray(_S2_NP, dtype=jnp.bfloat16)
    s3t = jnp.asarray(_S3T_NP, dtype=jnp.bfloat16)
    w_t = W_ext.reshape(_O, _K).T
    bias = b_ext.reshape(1, _O)

    out = pl.pallas_call(
        _conv_kernel,
        grid=(pl.cdiv(_H, _G),),
        in_specs=[
            pl.BlockSpec((1, 3, _G * _S, _W * _S), lambda n: (0, 0, n, 0)),
            pl.BlockSpec((_W * _S, _S * _LW), lambda n: (0, 0)),
            pl.BlockSpec((_K, _K), lambda n: (0, 0)),
            pl.BlockSpec((_K, _O), lambda n: (0, 0)),
            pl.BlockSpec((1, _O), lambda n: (0, 0)),
        ],
        out_specs=pl.BlockSpec((_G, _W, _O), lambda n: (n, 0, 0)),
        out_shape=jax.ShapeDtypeStruct((_H, _W, _O), jnp.float32),
        scratch_shapes=[pltpu.VMEM((_K, _O), jnp.bfloat16)],
    )(x, s2, s3t, w_t, bias)

    return out.transpose(2, 0, 1).reshape(1, _O, _H, _W)
